# baseline (speedup 1.0000x reference)
import functools

import jax
import jax.numpy as jnp
from jax import lax
from jax.experimental import pallas as pl
from jax.experimental.pallas import tpu as pltpu
from jax.experimental.pallas import tpu_sc as plsc

VOCAB = 1000000
DIM = 64
B = 16384
L = 20

NC = 2
NS = 16
NW = NC * NS

NTOT = B * L
PER_W = NTOT // NW
SUB = 128
NSUB = 4
CH = SUB * NSUB
NCHUNK = PER_W // CH

VCH = 128
NVCH = VOCAB // VCH
VTAIL = VOCAB - NVCH * VCH
VTAIL0 = NVCH * VCH
VPAD = VTAIL0 + VCH
VCH_W = -(-NVCH // NW)


def _tr_body(dbuf_c, dbuf_x, lines_c, lines_x, djvs, rvs):
    def row_body(vv, _):
        bvv = lax.broadcast(vv, (16,))
        base = vv * DIM
        for k in range(DIM // 16):
            vc = plsc.load_gather(dbuf_c, [djvs[k], rvs[k], bvv])
            lines_c[pl.ds(base + k * 16, 16)] = vc
            vx = plsc.load_gather(dbuf_x, [djvs[k], rvs[k], bvv])
            lines_x[pl.ds(base + k * 16, 16)] = vx
        return 0

    lax.fori_loop(0, VCH, row_body, 0)


def _tr_kernel(tin_hbm, tout_hbm, tailin_hbm, tailout_hbm, lin_hbm, lout_hbm,
               dbuf_c, dbuf_x, lines_c, lines_x, sem):
    wid = lax.axis_index("s") * NC + lax.axis_index("c")

    iota16 = lax.iota(jnp.int32, 16)
    djvs = []
    rvs = []
    for k in range(DIM // 16):
        d = k * 16 + iota16
        djvs.append(d >> 3)
        rvs.append(d & 7)

    def chunk_body(i, _):
        k = wid + i * NW

        @pl.when(k < NVCH)
        def _():
            v0 = pl.multiple_of(k * VCH, VCH)
            copies = []
            for dj in range(8):
                copies.append(pltpu.async_copy(
                    tin_hbm.at[pl.ds(dj * 8, 8), pl.ds(v0, VCH)],
                    dbuf_c.at[dj], sem))
                copies.append(pltpu.async_copy(
                    tout_hbm.at[pl.ds(dj * 8, 8), pl.ds(v0, VCH)],
                    dbuf_x.at[dj], sem))
            for cp in copies:
                cp.wait()

            _tr_body(dbuf_c, dbuf_x, lines_c, lines_x, djvs, rvs)

            pltpu.sync_copy(lines_c, lin_hbm.at[pl.ds(v0 * DIM, VCH * DIM)])
            pltpu.sync_copy(lines_x, lout_hbm.at[pl.ds(v0 * DIM, VCH * DIM)])
        return 0

    lax.fori_loop(0, VCH_W, chunk_body, 0)

    @pl.when(wid == NVCH % NW)
    def _():
        copies = []
        for dj in range(8):
            copies.append(pltpu.async_copy(
                tailin_hbm.at[pl.ds(dj * 8, 8)], dbuf_c.at[dj], sem))
            copies.append(pltpu.async_copy(
                tailout_hbm.at[pl.ds(dj * 8, 8)], dbuf_x.at[dj], sem))
        for cp in copies:
            cp.wait()

        _tr_body(dbuf_c, dbuf_x, lines_c, lines_x, djvs, rvs)

        pltpu.sync_copy(lines_c, lin_hbm.at[pl.ds(VTAIL0 * DIM, VCH * DIM)])
        pltpu.sync_copy(lines_x, lout_hbm.at[pl.ds(VTAIL0 * DIM, VCH * DIM)])


def _sc_kernel(cw_hbm, xw_hbm, in_hbm, out_emb_hbm, out_hbm,
               idx_c, idx_x, crows, xrows, scores, sem):
    wid = lax.axis_index("s") * NC + lax.axis_index("c")

    def chunk_body(c, _):
        pltpu.sync_copy(cw_hbm.at[wid, c], idx_c)
        pltpu.sync_copy(xw_hbm.at[wid, c], idx_x)

        copies = []
        for j in range(NSUB):
            copies.append(
                pltpu.async_copy(in_hbm.at[idx_c.at[j]], crows.at[j], sem))
            copies.append(
                pltpu.async_copy(out_emb_hbm.at[idx_x.at[j]], xrows.at[j], sem))
        for cp in copies:
            cp.wait()

        iota16 = lax.iota(jnp.int32, 16)
        for j in range(NSUB):
            def grp_body(g, _):
                out16 = jnp.zeros((16,), jnp.float32)
                for p in range(16):
                    i = g * 16 + p
                    acc = (crows[j, i, pl.ds(0, 16)] * xrows[j, i, pl.ds(0, 16)]
                           + crows[j, i, pl.ds(16, 16)] * xrows[j, i, pl.ds(16, 16)])
                    acc = acc + crows[j, i, pl.ds(32, 16)] * xrows[j, i, pl.ds(32, 16)]
                    acc = acc + crows[j, i, pl.ds(48, 16)] * xrows[j, i, pl.ds(48, 16)]
                    s = jnp.sum(acc)
                    out16 = jnp.where(iota16 == p, lax.broadcast(s, (16,)), out16)
                scores[j, pl.ds(g * 16, 16)] = out16
                return 0
            lax.fori_loop(0, SUB // 16, grp_body, 0)

        pltpu.sync_copy(scores, out_hbm.at[wid, c])
        return 0

    lax.fori_loop(0, NCHUNK, chunk_body, 0)


@jax.jit
def _run(cw, xw, tin, tout, tailin, tailout):
    mesh = plsc.VectorSubcoreMesh(core_axis_name="c", subcore_axis_name="s",
                                  num_cores=NC, num_subcores=NS)

    tr = pl.kernel(
        _tr_kernel,
        out_type=(jax.ShapeDtypeStruct((VPAD * DIM,), jnp.float32),
                  jax.ShapeDtypeStruct((VPAD * DIM,), jnp.float32)),
        mesh=mesh,
        compiler_params=pltpu.CompilerParams(needs_layout_passes=False,
                                             use_tc_tiling_on_sc=True),
        scratch_types=[
            pltpu.VMEM((8, 8, VCH), jnp.float32),
            pltpu.VMEM((8, 8, VCH), jnp.float32),
            pltpu.VMEM((VCH * DIM,), jnp.float32),
            pltpu.VMEM((VCH * DIM,), jnp.float32),
            pltpu.SemaphoreType.DMA,
        ],
    )
    lin, lout = tr(tin, tout, tailin, tailout)
    lin2 = lin.reshape(VPAD, DIM)
    lout2 = lout.reshape(VPAD, DIM)

    gd = pl.kernel(
        _sc_kernel,
        out_type=jax.ShapeDtypeStruct((NW, NCHUNK, NSUB, SUB), jnp.float32),
        mesh=mesh,
        compiler_params=pltpu.CompilerParams(needs_layout_passes=False,
                                             use_tc_tiling_on_sc=False),
        scratch_types=[
            pltpu.VMEM((NSUB, SUB), jnp.int32),
            pltpu.VMEM((NSUB, SUB), jnp.int32),
            pltpu.VMEM((NSUB, SUB, DIM), jnp.float32),
            pltpu.VMEM((NSUB, SUB, DIM), jnp.float32),
            pltpu.VMEM((NSUB, SUB), jnp.float32),
            pltpu.SemaphoreType.DMA,
        ],
    )
    return gd(cw, xw, lin2, lout2)


def kernel(center_words, context_words, in_embeddings, out_embeddings):
    cw = center_words.T.reshape(NW, NCHUNK, NSUB, SUB).astype(jnp.int32)
    xw = context_words.T.reshape(NW, NCHUNK, NSUB, SUB).astype(jnp.int32)
    tailin = jnp.pad(in_embeddings.T[:, VTAIL0:], ((0, 0), (0, VCH - VTAIL)))
    tailout = jnp.pad(out_embeddings.T[:, VTAIL0:], ((0, 0), (0, VCH - VTAIL)))
    scores = _run(cw, xw, in_embeddings.T, out_embeddings.T, tailin, tailout)
    return scores.reshape(L, B).T


# speedup vs baseline: 3.0853x; 3.0853x over previous
import functools

import jax
import jax.numpy as jnp
from jax import lax
from jax.experimental import pallas as pl
from jax.experimental.pallas import tpu as pltpu
from jax.experimental.pallas import tpu_sc as plsc

VOCAB = 1000000
DIM = 64
B = 16384
L = 20

NC = 2
NS = 16
NW = NC * NS

NTOT = B * L
PER_W = NTOT // NW
SUB = 128
NSUB = 2
CH = SUB * NSUB
NCHUNK = PER_W // CH

VSTEP = 512
GRID = 977
VCOV = 2 * GRID * VSTEP
HALF = VCOV // 2


def _tc_tr_kernel(a_lo, a_hi, b_lo, b_hi, o1_ref, o2_ref):
    o1_ref[...] = jnp.concatenate(
        [jnp.transpose(a_lo[...]), jnp.transpose(a_hi[...])], axis=1)
    o2_ref[...] = jnp.concatenate(
        [jnp.transpose(b_lo[...]), jnp.transpose(b_hi[...])], axis=1)


def _sc_kernel(kc_hbm, kx_hbm, oc_hbm, ox_hbm, in_hbm, out_emb_hbm, out_hbm,
               idx_c, idx_x, off_c, off_x, crows, xrows, scores, sem):
    wid = lax.axis_index("s") * NC + lax.axis_index("c")

    def chunk_body(c, _):
        pltpu.sync_copy(kc_hbm.at[wid, c], idx_c)
        pltpu.sync_copy(kx_hbm.at[wid, c], idx_x)
        pltpu.sync_copy(oc_hbm.at[wid, c], off_c)
        pltpu.sync_copy(ox_hbm.at[wid, c], off_x)

        copies = []
        for j in range(NSUB):
            copies.append(
                pltpu.async_copy(in_hbm.at[idx_c.at[j]], crows.at[j], sem))
            copies.append(
                pltpu.async_copy(out_emb_hbm.at[idx_x.at[j]], xrows.at[j], sem))
        for cp in copies:
            cp.wait()

        iota16 = lax.iota(jnp.int32, 16)
        for j in range(NSUB):
            def grp_body(g, _):
                out16 = jnp.zeros((16,), jnp.float32)
                ocv = off_c[j, pl.ds(g * 16, 16)]
                oxv = off_x[j, pl.ds(g * 16, 16)]
                for p in range(16):
                    i = g * 16 + p
                    poc = ocv[p]
                    pox = oxv[p]
                    acc = (crows[j, i, pl.ds(poc, 16)]
                           * xrows[j, i, pl.ds(pox, 16)]
                           + crows[j, i, pl.ds(poc + 16, 16)]
                           * xrows[j, i, pl.ds(pox + 16, 16)])
                    acc = acc + (crows[j, i, pl.ds(poc + 32, 16)]
                                 * xrows[j, i, pl.ds(pox + 32, 16)])
                    acc = acc + (crows[j, i, pl.ds(poc + 48, 16)]
                                 * xrows[j, i, pl.ds(pox + 48, 16)])
                    s = jnp.sum(acc)
                    out16 = jnp.where(iota16 == p, lax.broadcast(s, (16,)), out16)
                scores[j, pl.ds(g * 16, 16)] = out16
                return 0
            lax.fori_loop(0, SUB // 16, grp_body, 0)

        pltpu.sync_copy(scores, out_hbm.at[wid, c])
        return 0

    lax.fori_loop(0, NCHUNK, chunk_body, 0)


@jax.jit
def _run(cw, xw, tin, tout):
    tr = pl.pallas_call(
        _tc_tr_kernel,
        grid=(GRID,),
        in_specs=[pl.BlockSpec((DIM, VSTEP), lambda i: (0, i)),
                  pl.BlockSpec((DIM, VSTEP), lambda i: (0, i + GRID)),
                  pl.BlockSpec((DIM, VSTEP), lambda i: (0, i)),
                  pl.BlockSpec((DIM, VSTEP), lambda i: (0, i + GRID))],
        out_specs=[pl.BlockSpec((VSTEP, 2 * DIM), lambda i: (i, 0)),
                   pl.BlockSpec((VSTEP, 2 * DIM), lambda i: (i, 0))],
        out_shape=[jax.ShapeDtypeStruct((HALF, 2 * DIM), jnp.float32),
                   jax.ShapeDtypeStruct((HALF, 2 * DIM), jnp.float32)],
    )
    lin, lout = tr(tin, tin, tout, tout)

    mesh = plsc.VectorSubcoreMesh(core_axis_name="c", subcore_axis_name="s",
                                  num_cores=NC, num_subcores=NS)
    gd = pl.kernel(
        _sc_kernel,
        out_type=jax.ShapeDtypeStruct((NW, NCHUNK, NSUB, SUB), jnp.float32),
        mesh=mesh,
        compiler_params=pltpu.CompilerParams(needs_layout_passes=False,
                                             use_tc_tiling_on_sc=False),
        scratch_types=[
            pltpu.VMEM((NSUB, SUB), jnp.int32),
            pltpu.VMEM((NSUB, SUB), jnp.int32),
            pltpu.VMEM((NSUB, SUB), jnp.int32),
            pltpu.VMEM((NSUB, SUB), jnp.int32),
            pltpu.VMEM((NSUB, SUB, 2 * DIM), jnp.float32),
            pltpu.VMEM((NSUB, SUB, 2 * DIM), jnp.float32),
            pltpu.VMEM((NSUB, SUB), jnp.float32),
            pltpu.SemaphoreType.DMA,
        ],
    )
    kc = jnp.where(cw < HALF, cw, cw - HALF)
    kx = jnp.where(xw < HALF, xw, xw - HALF)
    oc = jnp.where(cw < HALF, 0, DIM).astype(jnp.int32)
    ox = jnp.where(xw < HALF, 0, DIM).astype(jnp.int32)
    return gd(kc, kx, oc, ox, lin, lout)


def kernel(center_words, context_words, in_embeddings, out_embeddings):
    cw = center_words.T.reshape(NW, NCHUNK, NSUB, SUB).astype(jnp.int32)
    xw = context_words.T.reshape(NW, NCHUNK, NSUB, SUB).astype(jnp.int32)
    scores = _run(cw, xw, in_embeddings.T, out_embeddings.T)
    return scores.reshape(L, B).T


# speedup vs baseline: 4.5709x; 1.4815x over previous
import functools

import jax
import jax.numpy as jnp
from jax import lax
from jax.experimental import pallas as pl
from jax.experimental.pallas import tpu as pltpu
from jax.experimental.pallas import tpu_sc as plsc

VOCAB = 1000000
DIM = 64
B = 16384
L = 20

NC = 2
NS = 16
NW = NC * NS

NTOT = B * L
PER_W = NTOT // NW
SUB = 128
NSUB = 2
CH = SUB * NSUB
NCHUNK = PER_W // CH

VSTEP = 2048
GRID = 245
VCOV = 2 * GRID * VSTEP
HALF = VCOV // 2
LASTB = (VOCAB - 1) // VSTEP


def _tc_tr_kernel(a_lo, a_hi, b_lo, b_hi, o1_ref, o2_ref):
    o1_ref[...] = jnp.concatenate(
        [jnp.transpose(a_lo[...]), jnp.transpose(a_hi[...])], axis=1)
    o2_ref[...] = jnp.concatenate(
        [jnp.transpose(b_lo[...]), jnp.transpose(b_hi[...])], axis=1)


def _sc_kernel(kc_hbm, kx_hbm, oc_hbm, ox_hbm, in_hbm, out_emb_hbm, out_hbm,
               idx_c, idx_x, off_c, off_x, crows, xrows, scores, sem):
    wid = lax.axis_index("s") * NC + lax.axis_index("c")

    def chunk_body(c, _):
        pltpu.sync_copy(kc_hbm.at[wid, c], idx_c)
        pltpu.sync_copy(kx_hbm.at[wid, c], idx_x)
        pltpu.sync_copy(oc_hbm.at[wid, c], off_c)
        pltpu.sync_copy(ox_hbm.at[wid, c], off_x)

        copies = []
        for j in range(NSUB):
            copies.append(
                pltpu.async_copy(in_hbm.at[idx_c.at[j]], crows.at[j], sem))
            copies.append(
                pltpu.async_copy(out_emb_hbm.at[idx_x.at[j]], xrows.at[j], sem))
        for cp in copies:
            cp.wait()

        iota16 = lax.iota(jnp.int32, 16)
        for j in range(NSUB):
            def grp_body(g, _):
                out16 = jnp.zeros((16,), jnp.float32)
                ocv = off_c[j, pl.ds(g * 16, 16)]
                oxv = off_x[j, pl.ds(g * 16, 16)]
                for p in range(16):
                    i = g * 16 + p
                    poc = ocv[p]
                    pox = oxv[p]
                    acc = (crows[j, i, pl.ds(poc, 16)]
                           * xrows[j, i, pl.ds(pox, 16)]
                           + crows[j, i, pl.ds(poc + 16, 16)]
                           * xrows[j, i, pl.ds(pox + 16, 16)])
                    acc = acc + (crows[j, i, pl.ds(poc + 32, 16)]
                                 * xrows[j, i, pl.ds(pox + 32, 16)])
                    acc = acc + (crows[j, i, pl.ds(poc + 48, 16)]
                                 * xrows[j, i, pl.ds(pox + 48, 16)])
                    s = jnp.sum(acc)
                    out16 = jnp.where(iota16 == p, lax.broadcast(s, (16,)), out16)
                scores[j, pl.ds(g * 16, 16)] = out16
                return 0
            lax.fori_loop(0, SUB // 16, grp_body, 0)

        pltpu.sync_copy(scores, out_hbm.at[wid, c])
        return 0

    lax.fori_loop(0, NCHUNK, chunk_body, 0)


@jax.jit
def _run(cw, xw, tin, tout):
    tr = pl.pallas_call(
        _tc_tr_kernel,
        grid=(GRID,),
        in_specs=[pl.BlockSpec((DIM, VSTEP), lambda i: (0, i)),
                  pl.BlockSpec((DIM, VSTEP),
                               lambda i: (0, jnp.minimum(i + GRID, LASTB))),
                  pl.BlockSpec((DIM, VSTEP), lambda i: (0, i)),
                  pl.BlockSpec((DIM, VSTEP),
                               lambda i: (0, jnp.minimum(i + GRID, LASTB)))],
        out_specs=[pl.BlockSpec((VSTEP, 2 * DIM), lambda i: (i, 0)),
                   pl.BlockSpec((VSTEP, 2 * DIM), lambda i: (i, 0))],
        out_shape=[jax.ShapeDtypeStruct((HALF, 2 * DIM), jnp.float32),
                   jax.ShapeDtypeStruct((HALF, 2 * DIM), jnp.float32)],
    )
    lin, lout = tr(tin, tin, tout, tout)

    mesh = plsc.VectorSubcoreMesh(core_axis_name="c", subcore_axis_name="s",
                                  num_cores=NC, num_subcores=NS)
    gd = pl.kernel(
        _sc_kernel,
        out_type=jax.ShapeDtypeStruct((NW, NCHUNK, NSUB, SUB), jnp.float32),
        mesh=mesh,
        compiler_params=pltpu.CompilerParams(needs_layout_passes=False,
                                             use_tc_tiling_on_sc=False),
        scratch_types=[
            pltpu.VMEM((NSUB, SUB), jnp.int32),
            pltpu.VMEM((NSUB, SUB), jnp.int32),
            pltpu.VMEM((NSUB, SUB), jnp.int32),
            pltpu.VMEM((NSUB, SUB), jnp.int32),
            pltpu.VMEM((NSUB, SUB, 2 * DIM), jnp.float32),
            pltpu.VMEM((NSUB, SUB, 2 * DIM), jnp.float32),
            pltpu.VMEM((NSUB, SUB), jnp.float32),
            pltpu.SemaphoreType.DMA,
        ],
    )
    kc = jnp.where(cw < HALF, cw, cw - HALF)
    kx = jnp.where(xw < HALF, xw, xw - HALF)
    oc = jnp.where(cw < HALF, 0, DIM).astype(jnp.int32)
    ox = jnp.where(xw < HALF, 0, DIM).astype(jnp.int32)
    return gd(kc, kx, oc, ox, lin, lout)


def kernel(center_words, context_words, in_embeddings, out_embeddings):
    cw = center_words.T.reshape(NW, NCHUNK, NSUB, SUB).astype(jnp.int32)
    xw = context_words.T.reshape(NW, NCHUNK, NSUB, SUB).astype(jnp.int32)
    scores = _run(cw, xw, in_embeddings.T, out_embeddings.T)
    return scores.reshape(L, B).T


# speedup vs baseline: 5.0427x; 1.1032x over previous
import functools

import jax
import jax.numpy as jnp
from jax import lax
from jax.experimental import pallas as pl
from jax.experimental.pallas import tpu as pltpu
from jax.experimental.pallas import tpu_sc as plsc

VOCAB = 1000000
DIM = 64
B = 16384
L = 20

NC = 2
NS = 16
NW = NC * NS

NTOT = B * L
PER_W = NTOT // NW
SUB = 128
NSUB = 2
CH = SUB * NSUB
NCHUNK = PER_W // CH

VSTEP = 4096
GRID = 123
VCOV = 2 * GRID * VSTEP
HALF = VCOV // 2
LASTB = (VOCAB - 1) // VSTEP


def _tc_tr_kernel(a_lo, a_hi, b_lo, b_hi, o1_ref, o2_ref):
    o1_ref[...] = jnp.concatenate(
        [jnp.transpose(a_lo[...]), jnp.transpose(a_hi[...])], axis=1)
    o2_ref[...] = jnp.concatenate(
        [jnp.transpose(b_lo[...]), jnp.transpose(b_hi[...])], axis=1)


def _sc_kernel(kc_hbm, kx_hbm, oc_hbm, ox_hbm, in_hbm, out_emb_hbm, out_hbm,
               idx_c, idx_x, off_c, off_x, crows, xrows, scores, sem):
    wid = lax.axis_index("s") * NC + lax.axis_index("c")

    def chunk_body(c, _):
        pltpu.sync_copy(kc_hbm.at[wid, c], idx_c)
        pltpu.sync_copy(kx_hbm.at[wid, c], idx_x)
        pltpu.sync_copy(oc_hbm.at[wid, c], off_c)
        pltpu.sync_copy(ox_hbm.at[wid, c], off_x)

        copies = []
        for j in range(NSUB):
            copies.append(
                pltpu.async_copy(in_hbm.at[idx_c.at[j]], crows.at[j], sem))
            copies.append(
                pltpu.async_copy(out_emb_hbm.at[idx_x.at[j]], xrows.at[j], sem))
        for cp in copies:
            cp.wait()

        iota16 = lax.iota(jnp.int32, 16)
        for j in range(NSUB):
            def grp_body(g, _):
                out16 = jnp.zeros((16,), jnp.float32)
                ocv = off_c[j, pl.ds(g * 16, 16)]
                oxv = off_x[j, pl.ds(g * 16, 16)]
                for p in range(16):
                    i = g * 16 + p
                    poc = ocv[p]
                    pox = oxv[p]
                    acc = (crows[j, i, pl.ds(poc, 16)]
                           * xrows[j, i, pl.ds(pox, 16)]
                           + crows[j, i, pl.ds(poc + 16, 16)]
                           * xrows[j, i, pl.ds(pox + 16, 16)])
                    acc = acc + (crows[j, i, pl.ds(poc + 32, 16)]
                                 * xrows[j, i, pl.ds(pox + 32, 16)])
                    acc = acc + (crows[j, i, pl.ds(poc + 48, 16)]
                                 * xrows[j, i, pl.ds(pox + 48, 16)])
                    s = jnp.sum(acc)
                    out16 = jnp.where(iota16 == p, lax.broadcast(s, (16,)), out16)
                scores[j, pl.ds(g * 16, 16)] = out16
                return 0
            lax.fori_loop(0, SUB // 16, grp_body, 0)

        pltpu.sync_copy(scores, out_hbm.at[wid, c])
        return 0

    lax.fori_loop(0, NCHUNK, chunk_body, 0)


@jax.jit
def _run(cw, xw, tin, tout):
    tr = pl.pallas_call(
        _tc_tr_kernel,
        grid=(GRID,),
        in_specs=[pl.BlockSpec((DIM, VSTEP), lambda i: (0, i)),
                  pl.BlockSpec((DIM, VSTEP),
                               lambda i: (0, jnp.minimum(i + GRID, LASTB))),
                  pl.BlockSpec((DIM, VSTEP), lambda i: (0, i)),
                  pl.BlockSpec((DIM, VSTEP),
                               lambda i: (0, jnp.minimum(i + GRID, LASTB)))],
        out_specs=[pl.BlockSpec((VSTEP, 2 * DIM), lambda i: (i, 0)),
                   pl.BlockSpec((VSTEP, 2 * DIM), lambda i: (i, 0))],
        out_shape=[jax.ShapeDtypeStruct((HALF, 2 * DIM), jnp.float32),
                   jax.ShapeDtypeStruct((HALF, 2 * DIM), jnp.float32)],
    )
    lin, lout = tr(tin, tin, tout, tout)

    mesh = plsc.VectorSubcoreMesh(core_axis_name="c", subcore_axis_name="s",
                                  num_cores=NC, num_subcores=NS)
    gd = pl.kernel(
        _sc_kernel,
        out_type=jax.ShapeDtypeStruct((NW, NCHUNK, NSUB, SUB), jnp.float32),
        mesh=mesh,
        compiler_params=pltpu.CompilerParams(needs_layout_passes=False,
                                             use_tc_tiling_on_sc=False),
        scratch_types=[
            pltpu.VMEM((NSUB, SUB), jnp.int32),
            pltpu.VMEM((NSUB, SUB), jnp.int32),
            pltpu.VMEM((NSUB, SUB), jnp.int32),
            pltpu.VMEM((NSUB, SUB), jnp.int32),
            pltpu.VMEM((NSUB, SUB, 2 * DIM), jnp.float32),
            pltpu.VMEM((NSUB, SUB, 2 * DIM), jnp.float32),
            pltpu.VMEM((NSUB, SUB), jnp.float32),
            pltpu.SemaphoreType.DMA,
        ],
    )
    kc = jnp.where(cw < HALF, cw, cw - HALF)
    kx = jnp.where(xw < HALF, xw, xw - HALF)
    oc = jnp.where(cw < HALF, 0, DIM).astype(jnp.int32)
    ox = jnp.where(xw < HALF, 0, DIM).astype(jnp.int32)
    return gd(kc, kx, oc, ox, lin, lout)


def kernel(center_words, context_words, in_embeddings, out_embeddings):
    cw = center_words.T.reshape(NW, NCHUNK, NSUB, SUB).astype(jnp.int32)
    xw = context_words.T.reshape(NW, NCHUNK, NSUB, SUB).astype(jnp.int32)
    scores = _run(cw, xw, in_embeddings.T, out_embeddings.T)
    return scores.reshape(L, B).T


# speedup vs baseline: 5.0814x; 1.0077x over previous
import functools

import jax
import jax.numpy as jnp
from jax import lax
from jax.experimental import pallas as pl
from jax.experimental.pallas import tpu as pltpu
from jax.experimental.pallas import tpu_sc as plsc

VOCAB = 1000000
DIM = 64
B = 16384
L = 20

NC = 2
NS = 16
NW = NC * NS

NTOT = B * L
PER_W = NTOT // NW
SUB = 128
NSUB = 2
CH = SUB * NSUB
NCHUNK = PER_W // CH

VSTEP = 8192
GRID = 62
VCOV = 2 * GRID * VSTEP
HALF = VCOV // 2
LASTB = (VOCAB - 1) // VSTEP


def _tc_tr_kernel(a_lo, a_hi, b_lo, b_hi, o1_ref, o2_ref):
    o1_ref[...] = jnp.concatenate(
        [jnp.transpose(a_lo[...]), jnp.transpose(a_hi[...])], axis=1)
    o2_ref[...] = jnp.concatenate(
        [jnp.transpose(b_lo[...]), jnp.transpose(b_hi[...])], axis=1)


def _sc_kernel(kc_hbm, kx_hbm, oc_hbm, ox_hbm, in_hbm, out_emb_hbm, out_hbm,
               idx_c, idx_x, off_c, off_x, crows, xrows, scores, sem):
    wid = lax.axis_index("s") * NC + lax.axis_index("c")

    def chunk_body(c, _):
        pltpu.sync_copy(kc_hbm.at[wid, c], idx_c)
        pltpu.sync_copy(kx_hbm.at[wid, c], idx_x)
        pltpu.sync_copy(oc_hbm.at[wid, c], off_c)
        pltpu.sync_copy(ox_hbm.at[wid, c], off_x)

        copies = []
        for j in range(NSUB):
            copies.append(
                pltpu.async_copy(in_hbm.at[idx_c.at[j]], crows.at[j], sem))
            copies.append(
                pltpu.async_copy(out_emb_hbm.at[idx_x.at[j]], xrows.at[j], sem))
        for cp in copies:
            cp.wait()

        iota16 = lax.iota(jnp.int32, 16)
        for j in range(NSUB):
            def grp_body(g, _):
                out16 = jnp.zeros((16,), jnp.float32)
                ocv = off_c[j, pl.ds(g * 16, 16)]
                oxv = off_x[j, pl.ds(g * 16, 16)]
                for p in range(16):
                    i = g * 16 + p
                    poc = ocv[p]
                    pox = oxv[p]
                    acc = (crows[j, i, pl.ds(poc, 16)]
                           * xrows[j, i, pl.ds(pox, 16)]
                           + crows[j, i, pl.ds(poc + 16, 16)]
                           * xrows[j, i, pl.ds(pox + 16, 16)])
                    acc = acc + (crows[j, i, pl.ds(poc + 32, 16)]
                                 * xrows[j, i, pl.ds(pox + 32, 16)])
                    acc = acc + (crows[j, i, pl.ds(poc + 48, 16)]
                                 * xrows[j, i, pl.ds(pox + 48, 16)])
                    s = jnp.sum(acc)
                    out16 = jnp.where(iota16 == p, lax.broadcast(s, (16,)), out16)
                scores[j, pl.ds(g * 16, 16)] = out16
                return 0
            lax.fori_loop(0, SUB // 16, grp_body, 0)

        pltpu.sync_copy(scores, out_hbm.at[wid, c])
        return 0

    lax.fori_loop(0, NCHUNK, chunk_body, 0)


@jax.jit
def _run(cw, xw, tin, tout):
    tr = pl.pallas_call(
        _tc_tr_kernel,
        grid=(GRID,),
        in_specs=[pl.BlockSpec((DIM, VSTEP), lambda i: (0, i)),
                  pl.BlockSpec((DIM, VSTEP),
                               lambda i: (0, jnp.minimum(i + GRID, LASTB))),
                  pl.BlockSpec((DIM, VSTEP), lambda i: (0, i)),
                  pl.BlockSpec((DIM, VSTEP),
                               lambda i: (0, jnp.minimum(i + GRID, LASTB)))],
        out_specs=[pl.BlockSpec((VSTEP, 2 * DIM), lambda i: (i, 0)),
                   pl.BlockSpec((VSTEP, 2 * DIM), lambda i: (i, 0))],
        out_shape=[jax.ShapeDtypeStruct((HALF, 2 * DIM), jnp.float32),
                   jax.ShapeDtypeStruct((HALF, 2 * DIM), jnp.float32)],
    )
    lin, lout = tr(tin, tin, tout, tout)

    mesh = plsc.VectorSubcoreMesh(core_axis_name="c", subcore_axis_name="s",
                                  num_cores=NC, num_subcores=NS)
    gd = pl.kernel(
        _sc_kernel,
        out_type=jax.ShapeDtypeStruct((NW, NCHUNK, NSUB, SUB), jnp.float32),
        mesh=mesh,
        compiler_params=pltpu.CompilerParams(needs_layout_passes=False,
                                             use_tc_tiling_on_sc=False),
        scratch_types=[
            pltpu.VMEM((NSUB, SUB), jnp.int32),
            pltpu.VMEM((NSUB, SUB), jnp.int32),
            pltpu.VMEM((NSUB, SUB), jnp.int32),
            pltpu.VMEM((NSUB, SUB), jnp.int32),
            pltpu.VMEM((NSUB, SUB, 2 * DIM), jnp.float32),
            pltpu.VMEM((NSUB, SUB, 2 * DIM), jnp.float32),
            pltpu.VMEM((NSUB, SUB), jnp.float32),
            pltpu.SemaphoreType.DMA,
        ],
    )
    kc = jnp.where(cw < HALF, cw, cw - HALF)
    kx = jnp.where(xw < HALF, xw, xw - HALF)
    oc = jnp.where(cw < HALF, 0, DIM).astype(jnp.int32)
    ox = jnp.where(xw < HALF, 0, DIM).astype(jnp.int32)
    return gd(kc, kx, oc, ox, lin, lout)


def kernel(center_words, context_words, in_embeddings, out_embeddings):
    cw = center_words.T.reshape(NW, NCHUNK, NSUB, SUB).astype(jnp.int32)
    xw = context_words.T.reshape(NW, NCHUNK, NSUB, SUB).astype(jnp.int32)
    scores = _run(cw, xw, in_embeddings.T, out_embeddings.T)
    return scores.reshape(L, B).T


# speedup vs baseline: 6.0704x; 1.1946x over previous
import functools

import jax
import jax.numpy as jnp
from jax import lax
from jax.experimental import pallas as pl
from jax.experimental.pallas import tpu as pltpu
from jax.experimental.pallas import tpu_sc as plsc

VOCAB = 1000000
DIM = 64
B = 16384
L = 20

NC = 2
NS = 16
NW = NC * NS

NTOT = B * L
PER_W = NTOT // NW
SUB = 64
NSUB = 2
CH = SUB * NSUB
NCHUNK = PER_W // CH

VSTEP = 8192
GRID = 62
VCOV = 2 * GRID * VSTEP
HALF = VCOV // 2
LASTB = (VOCAB - 1) // VSTEP


def _tc_tr_kernel(a_lo, a_hi, b_lo, b_hi, o1_ref, o2_ref):
    o1_ref[...] = jnp.concatenate(
        [jnp.transpose(a_lo[...]), jnp.transpose(a_hi[...])], axis=1)
    o2_ref[...] = jnp.concatenate(
        [jnp.transpose(b_lo[...]), jnp.transpose(b_hi[...])], axis=1)


def _sc_kernel(kc_hbm, kx_hbm, oc_hbm, ox_hbm, in_hbm, out_emb_hbm, out_hbm,
               idx_c, idx_x, off_c, off_x,
               crows0, xrows0, crows1, xrows1, scores0, scores1,
               sem0, sem1, osem0, osem1):
    wid = lax.axis_index("s") * NC + lax.axis_index("c")

    pltpu.sync_copy(kc_hbm.at[wid], idx_c)
    pltpu.sync_copy(kx_hbm.at[wid], idx_x)
    pltpu.sync_copy(oc_hbm.at[wid], off_c)
    pltpu.sync_copy(ox_hbm.at[wid], off_x)

    def fire(c, crows, xrows, sem):
        for j in range(NSUB):
            pltpu.async_copy(in_hbm.at[idx_c.at[c, j]], crows.at[j], sem)
            pltpu.async_copy(out_emb_hbm.at[idx_x.at[c, j]], xrows.at[j], sem)

    def drain(c, crows, xrows, sem):
        for j in range(NSUB):
            pltpu.make_async_copy(
                in_hbm.at[idx_c.at[c, j]], crows.at[j], sem).wait()
            pltpu.make_async_copy(
                out_emb_hbm.at[idx_x.at[c, j]], xrows.at[j], sem).wait()

    def compute(c, crows, xrows, scores):
        iota16 = lax.iota(jnp.int32, 16)
        for j in range(NSUB):
            def grp_body(g, _):
                out16 = jnp.zeros((16,), jnp.float32)
                ocv = off_c[c, j, pl.ds(g * 16, 16)]
                oxv = off_x[c, j, pl.ds(g * 16, 16)]
                for p in range(16):
                    i = g * 16 + p
                    poc = ocv[p]
                    pox = oxv[p]
                    acc = (crows[j, i, pl.ds(poc, 16)]
                           * xrows[j, i, pl.ds(pox, 16)]
                           + crows[j, i, pl.ds(poc + 16, 16)]
                           * xrows[j, i, pl.ds(pox + 16, 16)])
                    acc = acc + (crows[j, i, pl.ds(poc + 32, 16)]
                                 * xrows[j, i, pl.ds(pox + 32, 16)])
                    acc = acc + (crows[j, i, pl.ds(poc + 48, 16)]
                                 * xrows[j, i, pl.ds(pox + 48, 16)])
                    s = jnp.sum(acc)
                    out16 = jnp.where(iota16 == p, lax.broadcast(s, (16,)), out16)
                scores[j, pl.ds(g * 16, 16)] = out16
                return 0
            lax.fori_loop(0, SUB // 16, grp_body, 0)

    def owait(c, scores, osem):
        pltpu.make_async_copy(scores, out_hbm.at[wid, c], osem).wait()

    fire(0, crows0, xrows0, sem0)

    def pipe_body(c2, _):
        c0 = 2 * c2
        c1 = c0 + 1
        fire(c1, crows1, xrows1, sem1)
        drain(c0, crows0, xrows0, sem0)

        @pl.when(c2 > 0)
        def _():
            owait(c0 - 2, scores0, osem0)
        compute(c0, crows0, xrows0, scores0)
        pltpu.async_copy(scores0, out_hbm.at[wid, c0], osem0)

        @pl.when(c2 < NCHUNK // 2 - 1)
        def _():
            fire(c0 + 2, crows0, xrows0, sem0)
        drain(c1, crows1, xrows1, sem1)

        @pl.when(c2 > 0)
        def _():
            owait(c1 - 2, scores1, osem1)
        compute(c1, crows1, xrows1, scores1)
        pltpu.async_copy(scores1, out_hbm.at[wid, c1], osem1)
        return 0

    lax.fori_loop(0, NCHUNK // 2, pipe_body, 0)
    owait(NCHUNK - 2, scores0, osem0)
    owait(NCHUNK - 1, scores1, osem1)


@jax.jit
def _run(cw, xw, tin, tout):
    tr = pl.pallas_call(
        _tc_tr_kernel,
        grid=(GRID,),
        in_specs=[pl.BlockSpec((DIM, VSTEP), lambda i: (0, i)),
                  pl.BlockSpec((DIM, VSTEP),
                               lambda i: (0, jnp.minimum(i + GRID, LASTB))),
                  pl.BlockSpec((DIM, VSTEP), lambda i: (0, i)),
                  pl.BlockSpec((DIM, VSTEP),
                               lambda i: (0, jnp.minimum(i + GRID, LASTB)))],
        out_specs=[pl.BlockSpec((VSTEP, 2 * DIM), lambda i: (i, 0)),
                   pl.BlockSpec((VSTEP, 2 * DIM), lambda i: (i, 0))],
        out_shape=[jax.ShapeDtypeStruct((HALF, 2 * DIM), jnp.float32),
                   jax.ShapeDtypeStruct((HALF, 2 * DIM), jnp.float32)],
    )
    lin, lout = tr(tin, tin, tout, tout)

    mesh = plsc.VectorSubcoreMesh(core_axis_name="c", subcore_axis_name="s",
                                  num_cores=NC, num_subcores=NS)
    gd = pl.kernel(
        _sc_kernel,
        out_type=jax.ShapeDtypeStruct((NW, NCHUNK, NSUB, SUB), jnp.float32),
        mesh=mesh,
        compiler_params=pltpu.CompilerParams(needs_layout_passes=False,
                                             use_tc_tiling_on_sc=False),
        scratch_types=[
            pltpu.VMEM((NCHUNK, NSUB, SUB), jnp.int32),
            pltpu.VMEM((NCHUNK, NSUB, SUB), jnp.int32),
            pltpu.VMEM((NCHUNK, NSUB, SUB), jnp.int32),
            pltpu.VMEM((NCHUNK, NSUB, SUB), jnp.int32),
            pltpu.VMEM((NSUB, SUB, 2 * DIM), jnp.float32),
            pltpu.VMEM((NSUB, SUB, 2 * DIM), jnp.float32),
            pltpu.VMEM((NSUB, SUB, 2 * DIM), jnp.float32),
            pltpu.VMEM((NSUB, SUB, 2 * DIM), jnp.float32),
            pltpu.VMEM((NSUB, SUB), jnp.float32),
            pltpu.VMEM((NSUB, SUB), jnp.float32),
            pltpu.SemaphoreType.DMA,
            pltpu.SemaphoreType.DMA,
            pltpu.SemaphoreType.DMA,
            pltpu.SemaphoreType.DMA,
        ],
    )
    kc = jnp.where(cw < HALF, cw, cw - HALF)
    kx = jnp.where(xw < HALF, xw, xw - HALF)
    oc = jnp.where(cw < HALF, 0, DIM).astype(jnp.int32)
    ox = jnp.where(xw < HALF, 0, DIM).astype(jnp.int32)
    return gd(kc, kx, oc, ox, lin, lout)


def kernel(center_words, context_words, in_embeddings, out_embeddings):
    cw = center_words.T.reshape(NW, NCHUNK, NSUB, SUB).astype(jnp.int32)
    xw = context_words.T.reshape(NW, NCHUNK, NSUB, SUB).astype(jnp.int32)
    scores = _run(cw, xw, in_embeddings.T, out_embeddings.T)
    return scores.reshape(L, B).T
